# baseline (speedup 1.0000x reference)
import jax
import jax.numpy as jnp
from jax import lax
from jax.experimental import pallas as pl
from jax.experimental.pallas import tpu as pltpu
from jax.experimental.pallas import tpu_sc as plsc

SCORE_THR = 0.05
NMS_THR = 0.5
MAX_DET = 300
PAD_DET = 304
ACC_PAD = 320
NEG = -3e38

B, N, C = 8, 20000, 80
NSTR = 20480
NBLK = 79
NPAD = NBLK * 16 * 16
L1PAD = NBLK * 16
L2PAD = 80


def _score_body(cls_ref, s_ref, l_ref):
    c = cls_ref[0]
    m = jnp.max(c, axis=0)
    iot = lax.broadcasted_iota(jnp.int32, (C, N), 0)
    lbl = jnp.min(jnp.where(c == m[None, :], iot, C), axis=0)
    s_ref[pl.ds(0, N)] = m
    l_ref[pl.ds(0, N)] = lbl


def _scores_labels(cls_t):
    return pl.pallas_call(
        _score_body,
        grid=(B,),
        in_specs=[pl.BlockSpec((1, C, N), lambda b: (b, 0, 0))],
        out_specs=[
            pl.BlockSpec((NSTR,), lambda b: (b,)),
            pl.BlockSpec((NSTR,), lambda b: (b,)),
        ],
        out_shape=[
            jax.ShapeDtypeStruct((B * NSTR,), jnp.float32),
            jax.ShapeDtypeStruct((B * NSTR,), jnp.int32),
        ],
    )(cls_t)


def _sc_body(scores_hbm, boxes_hbm, labels_hbm,
             outb_hbm, outs_hbm, outl_hbm,
             s_v, x1_v, y1_v, x2_v, y2_v, lab_v,
             l1_v, l2_v,
             ax1_v, ay1_v, ax2_v, ay2_v,
             ob_v, os_v, ol_v):
    wid = lax.axis_index("s") * 2 + lax.axis_index("c")

    @pl.when(wid < B)
    def _():
        b = wid
        pltpu.sync_copy(scores_hbm.at[pl.ds(b * NSTR, N)],
                        s_v.at[pl.ds(0, N)])
        pltpu.sync_copy(boxes_hbm.at[pl.ds((b * 4 + 0) * N, N)], x1_v)
        pltpu.sync_copy(boxes_hbm.at[pl.ds((b * 4 + 1) * N, N)], y1_v)
        pltpu.sync_copy(boxes_hbm.at[pl.ds((b * 4 + 2) * N, N)], x2_v)
        pltpu.sync_copy(boxes_hbm.at[pl.ds((b * 4 + 3) * N, N)], y2_v)
        pltpu.sync_copy(labels_hbm.at[pl.ds(b * NSTR, N)], lab_v)

        iot = lax.iota(jnp.int32, 16)
        negv = jnp.full((16,), NEG, jnp.float32)
        m1f = jnp.full((16,), -1.0, jnp.float32)
        m1i = jnp.full((16,), -1, jnp.int32)
        zf = jnp.zeros((16,), jnp.float32)

        for k in range((NPAD - N) // 16):
            s_v[pl.ds(N + k * 16, 16)] = negv

        def init_out(i, _):
            ob_v[pl.ds(i * 16, 16)] = m1f
            return 0
        lax.fori_loop(0, (PAD_DET * 4) // 16, init_out, 0)

        def init_out2(i, _):
            os_v[pl.ds(i * 16, 16)] = m1f
            ol_v[pl.ds(i * 16, 16)] = m1i
            return 0
        lax.fori_loop(0, PAD_DET // 16, init_out2, 0)

        def init_acc(i, _):
            ax1_v[pl.ds(i * 16, 16)] = zf
            ay1_v[pl.ds(i * 16, 16)] = zf
            ax2_v[pl.ds(i * 16, 16)] = zf
            ay2_v[pl.ds(i * 16, 16)] = zf
            return 0
        lax.fori_loop(0, ACC_PAD // 16, init_acc, 0)

        def init_l1(blk, _):
            acc = negv
            for t in range(16):
                off = blk * 256 + t * 16
                acc = jnp.where(iot == t, jnp.max(s_v[pl.ds(off, 16)]), acc)
            l1_v[pl.ds(blk * 16, 16)] = acc
            return 0
        lax.fori_loop(0, NBLK, init_l1, 0)

        for v2 in range(L2PAD // 16):
            acc = negv
            for t in range(16):
                cc = v2 * 16 + t
                if cc < NBLK:
                    acc = jnp.where(iot == t,
                                    jnp.max(l1_v[pl.ds(cc * 16, 16)]), acc)
            l2_v[pl.ds(v2 * 16, 16)] = acc

        def pop_one():
            l2r = [l2_v[pl.ds(v * 16, 16)] for v in range(L2PAD // 16)]
            m = l2r[0]
            for v in range(1, L2PAD // 16):
                m = jnp.maximum(m, l2r[v])
            gm = jnp.max(m)
            valid = gm > jnp.float32(SCORE_THR)

            big = jnp.full((16,), jnp.int32(2**30))
            c2s = big
            for v in range(L2PAD // 16 - 1, -1, -1):
                f = plsc.all_reduce_ffs(l2r[v] == gm)
                c2s = jnp.where(f < 16, f + v * 16, c2s)
            c2s = jnp.minimum(c2s, jnp.int32(NBLK - 1))

            lv1 = plsc.load_gather(l1_v, [c2s * 16 + iot])
            lane1 = plsc.all_reduce_ffs(lv1 == gm)
            c1s = c2s * 16 + jnp.minimum(lane1, 15)

            sv = plsc.load_gather(s_v, [c1s * 16 + iot])
            lane0 = plsc.all_reduce_ffs(sv == gm)
            lane0 = jnp.minimum(lane0, 15)
            js = c1s * 16 + lane0

            snew = jnp.where(iot == lane0, NEG, sv)
            plsc.store_scatter(s_v, [c1s * 16 + iot], snew)
            lv1n = jnp.where(iot == lane1, jnp.max(snew), lv1)
            plsc.store_scatter(l1_v, [c2s * 16 + iot], lv1n)
            v2s = c2s // 16
            lane2 = c2s - v2s * 16
            l2c = plsc.load_gather(l2_v, [v2s * 16 + iot])
            l2n = jnp.where(iot == lane2, jnp.max(lv1n), l2c)
            plsc.store_scatter(l2_v, [v2s * 16 + iot], l2n)

            bx1 = plsc.load_gather(x1_v, [js])
            by1 = plsc.load_gather(y1_v, [js])
            bx2 = plsc.load_gather(x2_v, [js])
            by2 = plsc.load_gather(y2_v, [js])
            lj = plsc.load_gather(lab_v, [js])
            return gm, valid, bx1, by1, bx2, by2, lj

        def cond(state):
            nacc = state[0]
            pvalid = state[2]
            return (nacc < MAX_DET) & pvalid

        def body(state):
            nacc, _, _, pbx1, pby1, pbx2, pby2, plj = state
            pgm = state[1]

            cur = pop_one()

            pbarea = (pbx2 - pbx1) * (pby2 - pby1)
            sacc = iot < 0
            for k in range(PAD_DET // 16):
                axv = ax1_v[pl.ds(k * 16, 16)]
                ayv = ay1_v[pl.ds(k * 16, 16)]
                ax2v = ax2_v[pl.ds(k * 16, 16)]
                ay2v = ay2_v[pl.ds(k * 16, 16)]
                aav = (ax2v - axv) * (ay2v - ayv)
                xx1 = jnp.maximum(pbx1, axv)
                yy1 = jnp.maximum(pby1, ayv)
                xx2 = jnp.minimum(pbx2, ax2v)
                yy2 = jnp.minimum(pby2, ay2v)
                w = jnp.maximum(jnp.float32(0.0), xx2 - xx1)
                h = jnp.maximum(jnp.float32(0.0), yy2 - yy1)
                inter = w * h
                iou = inter / (aav + pbarea - inter + jnp.float32(1e-12))
                sacc = sacc | (iou > NMS_THR)
            accept = jnp.logical_not(jnp.any(sacc))

            @pl.when(accept)
            def _accept():
                ns = jnp.full((16,), nacc, jnp.int32)
                one0 = iot == 0
                plsc.store_scatter(ax1_v, [ns], pbx1, mask=one0)
                plsc.store_scatter(ay1_v, [ns], pby1, mask=one0)
                plsc.store_scatter(ax2_v, [ns], pbx2, mask=one0)
                plsc.store_scatter(ay2_v, [ns], pby2, mask=one0)
                plsc.store_scatter(os_v, [ns], jnp.full((16,), pgm),
                                   mask=one0)
                plsc.store_scatter(ol_v, [ns], plj, mask=one0)
                vb = jnp.where(iot == 1, pby1, pbx1)
                vb = jnp.where(iot == 2, pbx2, vb)
                vb = jnp.where(iot == 3, pby2, vb)
                plsc.store_scatter(ob_v, [ns * 4 + iot], vb, mask=iot < 4)

            nacc = jnp.where(accept, nacc + 1, nacc)
            cgm, cvalid, cbx1, cby1, cbx2, cby2, clj = cur
            return (nacc, cgm, cvalid, cbx1, cby1, cbx2, cby2, clj)

        first = pop_one()
        fgm, fvalid, fbx1, fby1, fbx2, fby2, flj = first
        lax.while_loop(cond, body,
                       (jnp.int32(0), fgm, fvalid,
                        fbx1, fby1, fbx2, fby2, flj))

        pltpu.sync_copy(ob_v, outb_hbm.at[pl.ds(b * PAD_DET * 4, PAD_DET * 4)])
        pltpu.sync_copy(os_v, outs_hbm.at[pl.ds(b * PAD_DET, PAD_DET)])
        pltpu.sync_copy(ol_v, outl_hbm.at[pl.ds(b * PAD_DET, PAD_DET)])


def _sc_nms(scores, boxes_t, labels):
    mesh = plsc.VectorSubcoreMesh(core_axis_name="c", subcore_axis_name="s")
    f32, i32 = jnp.float32, jnp.int32
    fn = pl.kernel(
        _sc_body,
        out_type=[
            jax.ShapeDtypeStruct((B * PAD_DET * 4,), f32),
            jax.ShapeDtypeStruct((B * PAD_DET,), f32),
            jax.ShapeDtypeStruct((B * PAD_DET,), i32),
        ],
        mesh=mesh,
        compiler_params=pltpu.CompilerParams(needs_layout_passes=False),
        scratch_types=[
            pltpu.VMEM((NPAD,), f32),
            pltpu.VMEM((N,), f32),
            pltpu.VMEM((N,), f32),
            pltpu.VMEM((N,), f32),
            pltpu.VMEM((N,), f32),
            pltpu.VMEM((N,), i32),
            pltpu.VMEM((L1PAD,), f32),
            pltpu.VMEM((L2PAD,), f32),
            pltpu.VMEM((ACC_PAD,), f32),
            pltpu.VMEM((ACC_PAD,), f32),
            pltpu.VMEM((ACC_PAD,), f32),
            pltpu.VMEM((ACC_PAD,), f32),
            pltpu.VMEM((PAD_DET * 4,), f32),
            pltpu.VMEM((PAD_DET,), f32),
            pltpu.VMEM((PAD_DET,), i32),
        ],
    )
    return fn(scores, boxes_t, labels)


def kernel(boxes, classification):
    cls_t = jnp.transpose(classification, (0, 2, 1))
    boxes_t = jnp.transpose(boxes, (0, 2, 1))
    scores, labels = _scores_labels(cls_t)
    boxes_f = boxes_t.reshape(B * 4 * N)
    outb, outs, outl = _sc_nms(scores, boxes_f, labels)
    fb = outb.reshape(B, PAD_DET, 4)[:, :MAX_DET]
    fs = outs.reshape(B, PAD_DET)[:, :MAX_DET]
    fl = outl.reshape(B, PAD_DET)[:, :MAX_DET]
    return fb, fs, fl


# speedup vs baseline: 1.0835x; 1.0835x over previous
import jax
import jax.numpy as jnp
from jax import lax
from jax.experimental import pallas as pl
from jax.experimental.pallas import tpu as pltpu
from jax.experimental.pallas import tpu_sc as plsc

SCORE_THR = 0.05
NMS_THR = 0.5
MAX_DET = 300
PAD_DET = 304
ACC_PAD = 320
NEG = -3e38

B, N, C = 8, 20000, 80
NSTR = 20480
NBLK = 79
NPAD = NBLK * 16 * 16
L1PAD = NBLK * 16
L2PAD = 80


def _score_body(cls_ref, s_ref, l_ref):
    c = cls_ref[0]
    m = jnp.max(c, axis=0)
    iot = lax.broadcasted_iota(jnp.int32, (C, N), 0)
    lbl = jnp.min(jnp.where(c == m[None, :], iot, C), axis=0)
    s_ref[pl.ds(0, N)] = m
    l_ref[pl.ds(0, N)] = lbl


def _scores_labels(cls_t):
    return pl.pallas_call(
        _score_body,
        grid=(B,),
        in_specs=[pl.BlockSpec((1, C, N), lambda b: (b, 0, 0))],
        out_specs=[
            pl.BlockSpec((NSTR,), lambda b: (b,)),
            pl.BlockSpec((NSTR,), lambda b: (b,)),
        ],
        out_shape=[
            jax.ShapeDtypeStruct((B * NSTR,), jnp.float32),
            jax.ShapeDtypeStruct((B * NSTR,), jnp.int32),
        ],
    )(cls_t)


def _sc_body(scores_hbm, boxes_hbm, labels_hbm,
             outb_hbm, outs_hbm, outl_hbm,
             s_v, x1_v, y1_v, x2_v, y2_v, lab_v,
             l1_v, l2_v,
             ax1_v, ay1_v, ax2_v, ay2_v,
             ob_v, os_v, ol_v):
    wid = lax.axis_index("s") * 2 + lax.axis_index("c")

    @pl.when(wid < B)
    def _():
        b = wid
        pltpu.sync_copy(scores_hbm.at[pl.ds(b * NSTR, N)],
                        s_v.at[pl.ds(0, N)])
        pltpu.sync_copy(boxes_hbm.at[pl.ds((b * 4 + 0) * N, N)], x1_v)
        pltpu.sync_copy(boxes_hbm.at[pl.ds((b * 4 + 1) * N, N)], y1_v)
        pltpu.sync_copy(boxes_hbm.at[pl.ds((b * 4 + 2) * N, N)], x2_v)
        pltpu.sync_copy(boxes_hbm.at[pl.ds((b * 4 + 3) * N, N)], y2_v)
        pltpu.sync_copy(labels_hbm.at[pl.ds(b * NSTR, N)], lab_v)

        iot = lax.iota(jnp.int32, 16)
        negv = jnp.full((16,), NEG, jnp.float32)
        m1f = jnp.full((16,), -1.0, jnp.float32)
        m1i = jnp.full((16,), -1, jnp.int32)
        zf = jnp.zeros((16,), jnp.float32)

        for k in range((NPAD - N) // 16):
            s_v[pl.ds(N + k * 16, 16)] = negv

        def init_out(i, _):
            ob_v[pl.ds(i * 16, 16)] = m1f
            return 0
        lax.fori_loop(0, (PAD_DET * 4) // 16, init_out, 0)

        def init_out2(i, _):
            os_v[pl.ds(i * 16, 16)] = m1f
            ol_v[pl.ds(i * 16, 16)] = m1i
            return 0
        lax.fori_loop(0, PAD_DET // 16, init_out2, 0)

        def init_acc(i, _):
            ax1_v[pl.ds(i * 16, 16)] = zf
            ay1_v[pl.ds(i * 16, 16)] = zf
            ax2_v[pl.ds(i * 16, 16)] = zf
            ay2_v[pl.ds(i * 16, 16)] = zf
            return 0
        lax.fori_loop(0, ACC_PAD // 16, init_acc, 0)

        def init_l1(blk, _):
            acc = negv
            for t in range(16):
                off = blk * 256 + t * 16
                acc = jnp.where(iot == t, jnp.max(s_v[pl.ds(off, 16)]), acc)
            l1_v[pl.ds(blk * 16, 16)] = acc
            return 0
        lax.fori_loop(0, NBLK, init_l1, 0)

        for v2 in range(L2PAD // 16):
            acc = negv
            for t in range(16):
                cc = v2 * 16 + t
                if cc < NBLK:
                    acc = jnp.where(iot == t,
                                    jnp.max(l1_v[pl.ds(cc * 16, 16)]), acc)
            l2_v[pl.ds(v2 * 16, 16)] = acc

        def pop_one():
            l2r = [l2_v[pl.ds(v * 16, 16)] for v in range(L2PAD // 16)]
            m = l2r[0]
            for v in range(1, L2PAD // 16):
                m = jnp.maximum(m, l2r[v])
            gm = jnp.max(m)
            valid = gm > jnp.float32(SCORE_THR)

            big = jnp.full((16,), jnp.int32(2**30))
            c2s = big
            for v in range(L2PAD // 16 - 1, -1, -1):
                f = plsc.all_reduce_ffs(l2r[v] == gm)
                c2s = jnp.where(f < 16, f + v * 16, c2s)
            c2s = jnp.minimum(c2s, jnp.int32(NBLK - 1))

            lv1 = plsc.load_gather(l1_v, [c2s * 16 + iot])
            lane1 = plsc.all_reduce_ffs(lv1 == gm)
            c1s = c2s * 16 + jnp.minimum(lane1, 15)

            sv = plsc.load_gather(s_v, [c1s * 16 + iot])
            lane0 = plsc.all_reduce_ffs(sv == gm)
            lane0 = jnp.minimum(lane0, 15)
            js = c1s * 16 + lane0

            snew = jnp.where(iot == lane0, NEG, sv)
            plsc.store_scatter(s_v, [c1s * 16 + iot], snew)
            lv1n = jnp.where(iot == lane1, jnp.max(snew), lv1)
            plsc.store_scatter(l1_v, [c2s * 16 + iot], lv1n)
            v2s = c2s // 16
            lane2 = c2s - v2s * 16
            l2c = plsc.load_gather(l2_v, [v2s * 16 + iot])
            l2n = jnp.where(iot == lane2, jnp.max(lv1n), l2c)
            plsc.store_scatter(l2_v, [v2s * 16 + iot], l2n)

            bx1 = plsc.load_gather(x1_v, [js])
            by1 = plsc.load_gather(y1_v, [js])
            bx2 = plsc.load_gather(x2_v, [js])
            by2 = plsc.load_gather(y2_v, [js])
            lj = plsc.load_gather(lab_v, [js])
            return gm, valid, bx1, by1, bx2, by2, lj

        def cond(state):
            nacc = state[0]
            pvalid = state[2]
            return (nacc < MAX_DET) & pvalid

        def body(state):
            nacc, _, _, pbx1, pby1, pbx2, pby2, plj = state
            pgm = state[1]

            cur = pop_one()

            pbarea = (pbx2 - pbx1) * (pby2 - pby1)
            nk2 = (nacc + 31) // 32

            def sbody(k2, sacc_c):
                for u in range(2):
                    k = k2 * 2 + u
                    axv = ax1_v[pl.ds(k * 16, 16)]
                    ayv = ay1_v[pl.ds(k * 16, 16)]
                    ax2v = ax2_v[pl.ds(k * 16, 16)]
                    ay2v = ay2_v[pl.ds(k * 16, 16)]
                    aav = (ax2v - axv) * (ay2v - ayv)
                    xx1 = jnp.maximum(pbx1, axv)
                    yy1 = jnp.maximum(pby1, ayv)
                    xx2 = jnp.minimum(pbx2, ax2v)
                    yy2 = jnp.minimum(pby2, ay2v)
                    w = jnp.maximum(jnp.float32(0.0), xx2 - xx1)
                    h = jnp.maximum(jnp.float32(0.0), yy2 - yy1)
                    inter = w * h
                    iou = inter / (aav + pbarea - inter + jnp.float32(1e-12))
                    sacc_c = sacc_c | (iou > NMS_THR)
                return sacc_c

            sacc = lax.fori_loop(0, nk2, sbody, iot < 0)
            accept = jnp.logical_not(jnp.any(sacc))

            @pl.when(accept)
            def _accept():
                ns = jnp.full((16,), nacc, jnp.int32)
                one0 = iot == 0
                plsc.store_scatter(ax1_v, [ns], pbx1, mask=one0)
                plsc.store_scatter(ay1_v, [ns], pby1, mask=one0)
                plsc.store_scatter(ax2_v, [ns], pbx2, mask=one0)
                plsc.store_scatter(ay2_v, [ns], pby2, mask=one0)
                plsc.store_scatter(os_v, [ns], jnp.full((16,), pgm),
                                   mask=one0)
                plsc.store_scatter(ol_v, [ns], plj, mask=one0)
                vb = jnp.where(iot == 1, pby1, pbx1)
                vb = jnp.where(iot == 2, pbx2, vb)
                vb = jnp.where(iot == 3, pby2, vb)
                plsc.store_scatter(ob_v, [ns * 4 + iot], vb, mask=iot < 4)

            nacc = jnp.where(accept, nacc + 1, nacc)
            cgm, cvalid, cbx1, cby1, cbx2, cby2, clj = cur
            return (nacc, cgm, cvalid, cbx1, cby1, cbx2, cby2, clj)

        first = pop_one()
        fgm, fvalid, fbx1, fby1, fbx2, fby2, flj = first
        lax.while_loop(cond, body,
                       (jnp.int32(0), fgm, fvalid,
                        fbx1, fby1, fbx2, fby2, flj))

        pltpu.sync_copy(ob_v, outb_hbm.at[pl.ds(b * PAD_DET * 4, PAD_DET * 4)])
        pltpu.sync_copy(os_v, outs_hbm.at[pl.ds(b * PAD_DET, PAD_DET)])
        pltpu.sync_copy(ol_v, outl_hbm.at[pl.ds(b * PAD_DET, PAD_DET)])


def _sc_nms(scores, boxes_t, labels):
    mesh = plsc.VectorSubcoreMesh(core_axis_name="c", subcore_axis_name="s")
    f32, i32 = jnp.float32, jnp.int32
    fn = pl.kernel(
        _sc_body,
        out_type=[
            jax.ShapeDtypeStruct((B * PAD_DET * 4,), f32),
            jax.ShapeDtypeStruct((B * PAD_DET,), f32),
            jax.ShapeDtypeStruct((B * PAD_DET,), i32),
        ],
        mesh=mesh,
        compiler_params=pltpu.CompilerParams(needs_layout_passes=False),
        scratch_types=[
            pltpu.VMEM((NPAD,), f32),
            pltpu.VMEM((N,), f32),
            pltpu.VMEM((N,), f32),
            pltpu.VMEM((N,), f32),
            pltpu.VMEM((N,), f32),
            pltpu.VMEM((N,), i32),
            pltpu.VMEM((L1PAD,), f32),
            pltpu.VMEM((L2PAD,), f32),
            pltpu.VMEM((ACC_PAD,), f32),
            pltpu.VMEM((ACC_PAD,), f32),
            pltpu.VMEM((ACC_PAD,), f32),
            pltpu.VMEM((ACC_PAD,), f32),
            pltpu.VMEM((PAD_DET * 4,), f32),
            pltpu.VMEM((PAD_DET,), f32),
            pltpu.VMEM((PAD_DET,), i32),
        ],
    )
    return fn(scores, boxes_t, labels)


def kernel(boxes, classification):
    cls_t = jnp.transpose(classification, (0, 2, 1))
    boxes_t = jnp.transpose(boxes, (0, 2, 1))
    scores, labels = _scores_labels(cls_t)
    boxes_f = boxes_t.reshape(B * 4 * N)
    outb, outs, outl = _sc_nms(scores, boxes_f, labels)
    fb = outb.reshape(B, PAD_DET, 4)[:, :MAX_DET]
    fs = outs.reshape(B, PAD_DET)[:, :MAX_DET]
    fl = outl.reshape(B, PAD_DET)[:, :MAX_DET]
    return fb, fs, fl
